# baseline (speedup 1.0000x reference)
import functools

import jax
import jax.numpy as jnp
import numpy as np
from jax.experimental import pallas as pl
from jax.experimental.pallas import tpu as pltpu

N_TOK = 32768
N_EMB = 8192
DIM = 256

BN = 1024
SB = 512


def _vq_body(b_ref, z_ref, w_ref, idx_ref, zq_ref):
    zf = z_ref[...]
    zq_ref[...] = zf
    a = jnp.sum(zf ** 2, axis=1, keepdims=True)
    z = (zf + zf).astype(jnp.bfloat16)
    bestv = None
    for t in range(N_EMB // SB):
        w = w_ref[pl.ds(t * SB, SB), :]
        s2 = jax.lax.dot_general(
            z, w,
            dimension_numbers=(((1,), (1,)), ((), ())),
            preferred_element_type=jnp.float32,
        )
        d = (a + b_ref[:, t * SB:(t + 1) * SB]) - s2

        base = t * (SB // 128)
        for c in range(SB // 128):
            dc = d[:, c * 128:(c + 1) * 128]
            if bestv is None:
                bestv, bestc = dc, jnp.zeros(dc.shape, jnp.int32)
            else:
                m = dc < bestv
                bestv = jnp.where(m, dc, bestv)
                bestc = jnp.where(m, base + c, bestc)

    lv = jnp.min(bestv, axis=1, keepdims=True)
    lane = jax.lax.broadcasted_iota(jnp.int32, bestv.shape, 1)
    cand = bestc * 128 + lane
    idx_ref[...] = jnp.min(jnp.where(bestv == lv, cand, N_EMB),
                           axis=1, keepdims=True)


def _vq_fused(z, Wb, b):
    n = z.shape[0]
    idx, z_q = pl.pallas_call(
        _vq_body,
        grid=(n // BN,),
        in_specs=[
            pl.BlockSpec((1, N_EMB), lambda i: (0, 0)),
            pl.BlockSpec((BN, DIM), lambda i: (i, 0)),
            pl.BlockSpec((N_EMB, DIM), lambda i: (0, 0)),
        ],
        out_specs=[
            pl.BlockSpec((BN, 1), lambda i: (i, 0)),
            pl.BlockSpec((BN, DIM), lambda i: (i, 0)),
        ],
        out_shape=[
            jax.ShapeDtypeStruct((n, 1), jnp.int32),
            jax.ShapeDtypeStruct((n, DIM), jnp.float32),
        ],
        compiler_params=pltpu.CompilerParams(
            dimension_semantics=("parallel",),
        ),
    )(b, z, Wb)
    return idx.reshape(n), z_q


def _one_device(z, W):
    b = jnp.sum(W ** 2, axis=1).reshape(1, N_EMB)
    indices, z_q = _vq_fused(z, W.astype(jnp.bfloat16), b)
    return (z_q, indices)


def kernel(z, W):
    devs = jax.devices()
    nd = max(d for d in (1, 2, 4, 8) if len(devs) >= d and N_TOK % (d * BN) == 0)
    if nd == 1:
        return _one_device(z, W)
    mesh = jax.sharding.Mesh(np.asarray(devs[:nd]), ("x",))
    P = jax.sharding.PartitionSpec
    fn = jax.shard_map(
        _one_device, mesh=mesh,
        in_specs=(P("x", None), P(None, None)),
        out_specs=(P("x", None), P("x")),
        check_vma=False,
    )
    return fn(z, W)


# speedup vs baseline: 2.1917x; 2.1917x over previous
import functools

import jax
import jax.numpy as jnp
import numpy as np
from jax.experimental import pallas as pl
from jax.experimental.pallas import tpu as pltpu

N_TOK = 32768
N_EMB = 8192
DIM = 256

BN = 1024
SB = 512


def _vq_body(b_ref, z_ref, w_ref, idx_ref, zq_ref):
    zf = z_ref[...]
    zq_ref[...] = zf
    a = jnp.sum(zf ** 2, axis=1, keepdims=True)
    z = (zf + zf).astype(jnp.bfloat16)
    bestv = None
    for t in range(N_EMB // SB):
        w = w_ref[pl.ds(t * SB, SB), :]
        s2 = jax.lax.dot_general(
            z, w,
            dimension_numbers=(((1,), (1,)), ((), ())),
            preferred_element_type=jnp.float32,
        )
        d = (a + b_ref[:, t * SB:(t + 1) * SB]) - s2

        base = t * (SB // 128)
        for c in range(SB // 128):
            dc = d[:, c * 128:(c + 1) * 128]
            if bestv is None:
                bestv, bestc = dc, jnp.zeros(dc.shape, jnp.int32)
            else:
                m = dc < bestv
                bestv = jnp.where(m, dc, bestv)
                bestc = jnp.where(m, base + c, bestc)

    lv = jnp.min(bestv, axis=1, keepdims=True)
    lane = jax.lax.broadcasted_iota(jnp.int32, bestv.shape, 1)
    cand = bestc * 128 + lane
    idx_ref[...] = jnp.min(jnp.where(bestv == lv, cand, N_EMB),
                           axis=1, keepdims=True)


def _vq_fused(z, Wb, b):
    n = z.shape[0]
    idx, z_q = pl.pallas_call(
        _vq_body,
        grid=(n // BN,),
        in_specs=[
            pl.BlockSpec((1, N_EMB), lambda i: (0, 0)),
            pl.BlockSpec((BN, DIM), lambda i: (i, 0)),
            pl.BlockSpec((N_EMB, DIM), lambda i: (0, 0)),
        ],
        out_specs=[
            pl.BlockSpec((BN, 1), lambda i: (i, 0)),
            pl.BlockSpec((BN, DIM), lambda i: (i, 0)),
        ],
        out_shape=[
            jax.ShapeDtypeStruct((n, 1), jnp.int32),
            jax.ShapeDtypeStruct((n, DIM), jnp.float32),
        ],
        compiler_params=pltpu.CompilerParams(
            dimension_semantics=("parallel",),
        ),
    )(b, z, Wb)
    return idx.reshape(n), z_q


def _one_device(z, W):
    b = jnp.sum(W ** 2, axis=1).reshape(1, N_EMB)
    indices, z_q = _vq_fused(z, W.astype(jnp.bfloat16), b)
    return (z_q, indices)


def kernel(z, W):
    return _one_device(z, W)


# speedup vs baseline: 2.2908x; 1.0452x over previous
import functools

import jax
import jax.numpy as jnp
import numpy as np
from jax.experimental import pallas as pl
from jax.experimental.pallas import tpu as pltpu

N_TOK = 32768
N_EMB = 8192
DIM = 256

BN = 2048
SB = 512


def _vq_body(b_ref, z_ref, w_ref, idx_ref, zq_ref):
    zf = z_ref[...]
    zq_ref[...] = zf
    a = jnp.sum(zf ** 2, axis=1, keepdims=True)
    z = (zf + zf).astype(jnp.bfloat16)
    bestv = None
    for t in range(N_EMB // SB):
        w = w_ref[pl.ds(t * SB, SB), :]
        s2 = jax.lax.dot_general(
            z, w,
            dimension_numbers=(((1,), (1,)), ((), ())),
            preferred_element_type=jnp.float32,
        )
        d = (a + b_ref[:, t * SB:(t + 1) * SB]) - s2

        base = t * (SB // 128)
        for c in range(SB // 128):
            dc = d[:, c * 128:(c + 1) * 128]
            if bestv is None:
                bestv, bestc = dc, jnp.zeros(dc.shape, jnp.int32)
            else:
                m = dc < bestv
                bestv = jnp.where(m, dc, bestv)
                bestc = jnp.where(m, base + c, bestc)

    lv = jnp.min(bestv, axis=1, keepdims=True)
    lane = jax.lax.broadcasted_iota(jnp.int32, bestv.shape, 1)
    cand = bestc * 128 + lane
    idx_ref[...] = jnp.min(jnp.where(bestv == lv, cand, N_EMB),
                           axis=1, keepdims=True)


def _vq_fused(z, Wb, b):
    n = z.shape[0]
    idx, z_q = pl.pallas_call(
        _vq_body,
        grid=(n // BN,),
        in_specs=[
            pl.BlockSpec((1, N_EMB), lambda i: (0, 0)),
            pl.BlockSpec((BN, DIM), lambda i: (i, 0)),
            pl.BlockSpec((N_EMB, DIM), lambda i: (0, 0)),
        ],
        out_specs=[
            pl.BlockSpec((BN, 1), lambda i: (i, 0)),
            pl.BlockSpec((BN, DIM), lambda i: (i, 0)),
        ],
        out_shape=[
            jax.ShapeDtypeStruct((n, 1), jnp.int32),
            jax.ShapeDtypeStruct((n, DIM), jnp.float32),
        ],
        compiler_params=pltpu.CompilerParams(
            dimension_semantics=("parallel",),
        ),
    )(b, z, Wb)
    return idx.reshape(n), z_q


def _one_device(z, W):
    b = jnp.sum(W ** 2, axis=1).reshape(1, N_EMB)
    indices, z_q = _vq_fused(z, W.astype(jnp.bfloat16), b)
    return (z_q, indices)


def kernel(z, W):
    return _one_device(z, W)


# speedup vs baseline: 2.3118x; 1.0091x over previous
import functools

import jax
import jax.numpy as jnp
import numpy as np
from jax.experimental import pallas as pl
from jax.experimental.pallas import tpu as pltpu

N_TOK = 32768
N_EMB = 8192
DIM = 256

BN = 4096
SB = 512


def _vq_body(b_ref, z_ref, w_ref, idx_ref, zq_ref):
    zf = z_ref[...]
    zq_ref[...] = zf
    a = jnp.sum(zf ** 2, axis=1, keepdims=True)
    z = (zf + zf).astype(jnp.bfloat16)
    bestv = None
    for t in range(N_EMB // SB):
        w = w_ref[pl.ds(t * SB, SB), :]
        s2 = jax.lax.dot_general(
            z, w,
            dimension_numbers=(((1,), (1,)), ((), ())),
            preferred_element_type=jnp.float32,
        )
        d = (a + b_ref[:, t * SB:(t + 1) * SB]) - s2

        base = t * (SB // 128)
        for c in range(SB // 128):
            dc = d[:, c * 128:(c + 1) * 128]
            if bestv is None:
                bestv, bestc = dc, jnp.zeros(dc.shape, jnp.int32)
            else:
                m = dc < bestv
                bestv = jnp.where(m, dc, bestv)
                bestc = jnp.where(m, base + c, bestc)

    lv = jnp.min(bestv, axis=1, keepdims=True)
    lane = jax.lax.broadcasted_iota(jnp.int32, bestv.shape, 1)
    cand = bestc * 128 + lane
    idx_ref[...] = jnp.min(jnp.where(bestv == lv, cand, N_EMB),
                           axis=1, keepdims=True)


def _vq_fused(z, Wb, b):
    n = z.shape[0]
    idx, z_q = pl.pallas_call(
        _vq_body,
        grid=(n // BN,),
        in_specs=[
            pl.BlockSpec((1, N_EMB), lambda i: (0, 0)),
            pl.BlockSpec((BN, DIM), lambda i: (i, 0)),
            pl.BlockSpec((N_EMB, DIM), lambda i: (0, 0)),
        ],
        out_specs=[
            pl.BlockSpec((BN, 1), lambda i: (i, 0)),
            pl.BlockSpec((BN, DIM), lambda i: (i, 0)),
        ],
        out_shape=[
            jax.ShapeDtypeStruct((n, 1), jnp.int32),
            jax.ShapeDtypeStruct((n, DIM), jnp.float32),
        ],
        compiler_params=pltpu.CompilerParams(
            dimension_semantics=("parallel",),
        ),
    )(b, z, Wb)
    return idx.reshape(n), z_q


def _one_device(z, W):
    b = jnp.sum(W ** 2, axis=1).reshape(1, N_EMB)
    indices, z_q = _vq_fused(z, W.astype(jnp.bfloat16), b)
    return (z_q, indices)


def kernel(z, W):
    return _one_device(z, W)


# speedup vs baseline: 2.3149x; 1.0014x over previous
import functools

import jax
import jax.numpy as jnp
import numpy as np
from jax.experimental import pallas as pl
from jax.experimental.pallas import tpu as pltpu

N_TOK = 32768
N_EMB = 8192
DIM = 256

BN = 4096
SB = 1024


def _vq_body(b_ref, z_ref, w_ref, idx_ref, zq_ref):
    zf = z_ref[...]
    zq_ref[...] = zf
    a = jnp.sum(zf ** 2, axis=1, keepdims=True)
    z = (zf + zf).astype(jnp.bfloat16)
    bestv = None
    for t in range(N_EMB // SB):
        w = w_ref[pl.ds(t * SB, SB), :]
        s2 = jax.lax.dot_general(
            z, w,
            dimension_numbers=(((1,), (1,)), ((), ())),
            preferred_element_type=jnp.float32,
        )
        d = (a + b_ref[:, t * SB:(t + 1) * SB]) - s2

        base = t * (SB // 128)
        for c in range(SB // 128):
            dc = d[:, c * 128:(c + 1) * 128]
            if bestv is None:
                bestv, bestc = dc, jnp.zeros(dc.shape, jnp.int32)
            else:
                m = dc < bestv
                bestv = jnp.where(m, dc, bestv)
                bestc = jnp.where(m, base + c, bestc)

    lv = jnp.min(bestv, axis=1, keepdims=True)
    lane = jax.lax.broadcasted_iota(jnp.int32, bestv.shape, 1)
    cand = bestc * 128 + lane
    idx_ref[...] = jnp.min(jnp.where(bestv == lv, cand, N_EMB),
                           axis=1, keepdims=True)


def _vq_fused(z, Wb, b):
    n = z.shape[0]
    idx, z_q = pl.pallas_call(
        _vq_body,
        grid=(n // BN,),
        in_specs=[
            pl.BlockSpec((1, N_EMB), lambda i: (0, 0)),
            pl.BlockSpec((BN, DIM), lambda i: (i, 0)),
            pl.BlockSpec((N_EMB, DIM), lambda i: (0, 0)),
        ],
        out_specs=[
            pl.BlockSpec((BN, 1), lambda i: (i, 0)),
            pl.BlockSpec((BN, DIM), lambda i: (i, 0)),
        ],
        out_shape=[
            jax.ShapeDtypeStruct((n, 1), jnp.int32),
            jax.ShapeDtypeStruct((n, DIM), jnp.float32),
        ],
        compiler_params=pltpu.CompilerParams(
            dimension_semantics=("parallel",),
        ),
    )(b, z, Wb)
    return idx.reshape(n), z_q


def _one_device(z, W):
    b = jnp.sum(W ** 2, axis=1).reshape(1, N_EMB)
    indices, z_q = _vq_fused(z, W.astype(jnp.bfloat16), b)
    return (z_q, indices)


def kernel(z, W):
    return _one_device(z, W)
